# baseline (speedup 1.0000x reference)
import functools

import jax
import jax.numpy as jnp
from jax import lax
from jax.experimental import pallas as pl
from jax.experimental.pallas import tpu as pltpu
from jax.experimental.pallas import tpu_sc as plsc

N, P2, W2, TOPK, CKV = 8, 49, 64, 4, 192
R = N * P2 * W2
B = R * TOPK
NWIN = N * P2
NC, NS, L = 2, 16, 16
NW = NC * NS
MAXWIN = -(-NWIN // NW)
NPAIR = -(-MAXWIN // 2)
QB = W2 // 4
VJ = CKV // L
ISTRIDE = 512

_mesh = plsc.VectorSubcoreMesh(core_axis_name="c", subcore_axis_name="s")


@functools.partial(
    pl.kernel,
    mesh=_mesh,
    compiler_params=pltpu.CompilerParams(use_tc_tiling_on_sc=True),
    out_type=jax.ShapeDtypeStruct((R, TOPK, CKV), jnp.float32),
    scratch_types=[
        pltpu.VMEM((MAXWIN * ISTRIDE,), jnp.int32),
        [pltpu.VMEM((W2, CKV), jnp.float32)] * 2,
        [pltpu.VMEM((QB, TOPK, CKV), jnp.float32)] * 4,
        pltpu.SemaphoreType.DMA,
        [pltpu.SemaphoreType.DMA] * 2,
        [pltpu.SemaphoreType.DMA] * 4,
    ],
)
def _sc_gather(idx_hbm, kv_hbm, out_hbm, idx_v, win, wbuf, isem, ssems, wsems):
    wid = lax.axis_index("s") * NC + lax.axis_index("c")
    nwin = jnp.where(wid < NWIN - (MAXWIN - 1) * NW, MAXWIN, MAXWIN - 1)

    ih = []
    for k in range(MAXWIN):
        woff = jnp.minimum(wid + k * NW, NWIN - 1)
        ih.append(pltpu.async_copy(
            idx_hbm.at[pl.ds(woff * (W2 * TOPK), W2 * TOPK)],
            idx_v.at[pl.ds(k * ISTRIDE, W2 * TOPK)], isem))

    def stage_descr(k, par):
        return pltpu.make_async_copy(kv_hbm.at[wid + k * NW], win[par], ssems[par])

    def write_descr(k, q):
        wrow = (wid + k * NW) * W2 + q * QB
        return pltpu.make_async_copy(wbuf[q], out_hbm.at[pl.ds(wrow, QB)],
                                     wsems[q])

    stage_descr(0, 0).start()
    for h in ih:
        h.wait()

    def do_window(k, par):
        @pl.when(k + 1 < nwin)
        def _():
            stage_descr(k + 1, 1 - par).start()

        stage_descr(k, par).wait()

        for q in range(4):
            @pl.when(k >= 1)
            def _():
                write_descr(k - 1, q).wait()

            @plsc.parallel_loop(0, QB, unroll=4)
            def block(b):
                iv = idx_v[pl.ds(k * ISTRIDE + (q * QB + b) * TOPK, L)]
                for t in range(TOPK):
                    r = iv[t]
                    for j in range(VJ):
                        wbuf[q][b, t, pl.ds(j * L, L)] = win[par][r, pl.ds(j * L, L)]

            write_descr(k, q).start()

    def pair(kk, _):
        for par in range(2):
            k = kk * 2 + par

            @pl.when(k < nwin)
            def _():
                do_window(k, par)
        return 0

    lax.fori_loop(0, NPAIR, pair, 0)

    for q in range(4):
        write_descr(0, q).wait()


def kernel(r_idx, r_weight, kv):
    del r_weight
    idx_flat = r_idx.reshape(B)
    kv3 = kv.reshape(NWIN, W2, CKV)
    out3 = _sc_gather(idx_flat, kv3)
    return out3.reshape(N, P2, W2, TOPK, CKV)


# speedup vs baseline: 1.1316x; 1.1316x over previous
import functools

import jax
import jax.numpy as jnp
from jax import lax
from jax.experimental import pallas as pl
from jax.experimental.pallas import tpu as pltpu
from jax.experimental.pallas import tpu_sc as plsc

N, P2, W2, TOPK, CKV = 8, 49, 64, 4, 192
R = N * P2 * W2
B = R * TOPK
NWIN = N * P2
NC, NS, L = 2, 16, 16
NW = NC * NS
MAXWIN = -(-NWIN // NW)
NPAIR = -(-MAXWIN // 2)
QB = W2 // 4
VJ = CKV // L
ISTRIDE = 512

_mesh = plsc.VectorSubcoreMesh(core_axis_name="c", subcore_axis_name="s")


@functools.partial(
    pl.kernel,
    mesh=_mesh,
    compiler_params=pltpu.CompilerParams(use_tc_tiling_on_sc=True,
                                         needs_layout_passes=False),
    out_type=jax.ShapeDtypeStruct((R, TOPK, CKV), jnp.float32),
    scratch_types=[
        pltpu.VMEM((MAXWIN, TOPK, W2), jnp.int32),
        [pltpu.VMEM((W2, CKV), jnp.float32)] * 2,
        [pltpu.VMEM((QB, TOPK, CKV), jnp.float32)] * 4,
        pltpu.SemaphoreType.DMA,
        [pltpu.SemaphoreType.DMA] * 2,
        [pltpu.SemaphoreType.DMA] * 4,
    ],
)
def _sc_gather(idx_hbm, kv_hbm, out_hbm, idx_v, win, wbuf, isem, ssems, wsems):
    wid = lax.axis_index("s") * NC + lax.axis_index("c")
    nwin = jnp.where(wid < NWIN - (MAXWIN - 1) * NW, MAXWIN, MAXWIN - 1)

    ih = []
    for k in range(MAXWIN):
        woff = jnp.minimum(wid + k * NW, NWIN - 1)
        ih.append(pltpu.async_copy(idx_hbm.at[woff], idx_v.at[k], isem))

    def stage_descr(k, par):
        return pltpu.make_async_copy(kv_hbm.at[wid + k * NW], win[par], ssems[par])

    def write_descr(k, q):
        wrow = (wid + k * NW) * W2 + q * QB
        return pltpu.make_async_copy(wbuf[q], out_hbm.at[pl.ds(wrow, QB)],
                                     wsems[q])

    stage_descr(0, 0).start()
    for h in ih:
        h.wait()

    lane = lax.iota(jnp.int32, L)
    tvec = lane & (TOPK - 1)

    def do_window(k, par):
        @pl.when(k + 1 < nwin)
        def _():
            stage_descr(k + 1, 1 - par).start()

        stage_descr(k, par).wait()
        kvec = jnp.full((L,), k, jnp.int32)

        for q in range(4):
            @pl.when(k >= 1)
            def _():
                write_descr(k - 1, q).wait()

            @plsc.parallel_loop(0, QB, unroll=2)
            def block(b):
                bvec = jnp.full((L,), q * QB + b, jnp.int32)
                iv = plsc.load_gather(idx_v, [kvec, tvec, bvec])
                for t in range(TOPK):
                    r = iv[t]
                    for j in range(VJ):
                        wbuf[q][b, t, pl.ds(j * L, L)] = win[par][r, pl.ds(j * L, L)]

            write_descr(k, q).start()

    def pair(kk, _):
        for par in range(2):
            k = kk * 2 + par

            @pl.when(k < nwin)
            def _():
                do_window(k, par)
        return 0

    lax.fori_loop(0, NPAIR, pair, 0)

    for q in range(4):
        write_descr(0, q).wait()


def kernel(r_idx, r_weight, kv):
    del r_weight
    idx3 = jnp.transpose(r_idx, (0, 1, 3, 2)).reshape(NWIN, TOPK, W2)
    kv3 = kv.reshape(NWIN, W2, CKV)
    out3 = _sc_gather(idx3, kv3)
    return out3.reshape(N, P2, W2, TOPK, CKV)
